# baseline (speedup 1.0000x reference)
import functools

import jax
import jax.numpy as jnp
from jax import lax
from jax.experimental import pallas as pl
from jax.experimental.pallas import tpu as pltpu
from jax.experimental.pallas import tpu_sc as plsc

DIM_VOCAB = 100000
DIM_HIDDEN = 64
BATCH = 4096
HIST_LEN = 50

NUM_WORKERS = 32
B_PER_W = BATCH // NUM_WORKERS
NBUF = 8
N_ROUNDS = B_PER_W // NBUF

_mesh = plsc.VectorSubcoreMesh(core_axis_name="c", subcore_axis_name="s")


@functools.partial(
    pl.kernel,
    out_type=jax.ShapeDtypeStruct((BATCH, HIST_LEN, DIM_HIDDEN), jnp.float32),
    mesh=_mesh,
    scratch_types=[
        pltpu.VMEM((B_PER_W, HIST_LEN), jnp.int32),
        [pltpu.VMEM((HIST_LEN, DIM_HIDDEN), jnp.float32) for _ in range(NBUF)],
        [pltpu.SemaphoreType.DMA for _ in range(NBUF)],
        [pltpu.SemaphoreType.DMA for _ in range(NBUF)],
    ],
    compiler_params=pltpu.CompilerParams(use_tc_tiling_on_sc=False),
)
def _embed_lookup(idx_hbm, table_hbm, out_hbm, idx_v, rows, gsem, osem):
    wid = lax.axis_index("s") * 2 + lax.axis_index("c")
    base = wid * B_PER_W
    pltpu.sync_copy(idx_hbm.at[wid], idx_v)

    def gather(c, b):
        return pltpu.make_async_copy(
            table_hbm.at[idx_v.at[c]], rows[b], gsem[b])

    def put(c, b):
        return pltpu.make_async_copy(rows[b], out_hbm.at[base + c], osem[b])

    def body(g, carry):
        for b in range(NBUF):
            c = g * NBUF + b
            @pl.when(g > 0)
            def _():
                put(c - NBUF, b).wait()
            gather(c, b).start()
        for b in range(NBUF):
            c = g * NBUF + b
            gather(c, b).wait()
            put(c, b).start()
        return carry

    lax.fori_loop(0, N_ROUNDS, body, 0)
    for b in range(NBUF):
        put(B_PER_W - NBUF + b, b).wait()


def kernel(x, emb_t):
    idx = x.reshape(NUM_WORKERS, B_PER_W, HIST_LEN).astype(jnp.int32)
    return _embed_lookup(idx, emb_t)


# speedup vs baseline: 1.0746x; 1.0746x over previous
import functools

import jax
import jax.numpy as jnp
from jax import lax
from jax.experimental import pallas as pl
from jax.experimental.pallas import tpu as pltpu
from jax.experimental.pallas import tpu_sc as plsc

DIM_VOCAB = 100000
DIM_HIDDEN = 64
PAD_DIM = 128
BATCH = 4096
HIST_LEN = 50

NUM_WORKERS = 32
B_PER_W = BATCH // NUM_WORKERS
NBUF = 4
N_ROUNDS = B_PER_W // NBUF

_mesh = plsc.VectorSubcoreMesh(core_axis_name="c", subcore_axis_name="s")


@functools.partial(
    pl.kernel,
    out_type=jax.ShapeDtypeStruct((BATCH, HIST_LEN, DIM_HIDDEN), jnp.float32),
    mesh=_mesh,
    scratch_types=[
        pltpu.VMEM((B_PER_W, HIST_LEN), jnp.int32),
        [pltpu.VMEM((HIST_LEN, PAD_DIM), jnp.float32) for _ in range(NBUF)],
        [pltpu.VMEM((HIST_LEN, DIM_HIDDEN), jnp.float32) for _ in range(NBUF)],
        [pltpu.SemaphoreType.DMA for _ in range(NBUF)],
        [pltpu.SemaphoreType.DMA for _ in range(NBUF)],
    ],
    compiler_params=pltpu.CompilerParams(use_tc_tiling_on_sc=True),
)
def _embed_lookup(idx_hbm, table_hbm, out_hbm, idx_v, rows, pk, gsem, osem):
    wid = lax.axis_index("s") * 2 + lax.axis_index("c")
    base = wid * B_PER_W
    pltpu.sync_copy(idx_hbm.at[pl.ds(base, B_PER_W)], idx_v)

    def gather(c, b):
        return pltpu.make_async_copy(
            table_hbm.at[idx_v.at[c]], rows[b], gsem[b])

    def put(c, b):
        return pltpu.make_async_copy(pk[b], out_hbm.at[base + c], osem[b])

    def repack(b):
        def row(t, carry):
            for k in range(DIM_HIDDEN // 16):
                pk[b][t, pl.ds(k * 16, 16)] = rows[b][t, pl.ds(k * 16, 16)]
            return carry
        lax.fori_loop(0, HIST_LEN, row, 0)

    def body(g, carry):
        for b in range(NBUF):
            c = g * NBUF + b
            @pl.when(g > 0)
            def _():
                put(c - NBUF, b).wait()
            gather(c, b).start()
        for b in range(NBUF):
            c = g * NBUF + b
            gather(c, b).wait()
            repack(b)
            put(c, b).start()
        return carry

    lax.fori_loop(0, N_ROUNDS, body, 0)
    for b in range(NBUF):
        put(B_PER_W - NBUF + b, b).wait()


def kernel(x, emb_t):
    table = jnp.pad(emb_t, ((0, 0), (0, PAD_DIM - DIM_HIDDEN)))
    return _embed_lookup(x.astype(jnp.int32), table)
